# Initial kernel scaffold; baseline (speedup 1.0000x reference)
import jax
import jax.numpy as jnp
from jax.experimental import pallas as pl


def kernel(z, codebook):
    raise NotImplementedError("write your pallas kernel here")


# speedup vs baseline: 1.0530x; 1.0530x over previous
import functools

import jax
import jax.numpy as jnp
from jax import lax
from jax.experimental import pallas as pl
from jax.experimental.pallas import tpu as pltpu
from jax.experimental.pallas import tpu_sc as plsc


def _argmin_body(x_ref, cb_ref, xsq_ref, esq_ref, idx_ref, loss_ref,
                 runmin_ref, runidx_ref, *, block_k, n_elems):
    k = pl.program_id(0)
    nk = pl.num_programs(0)

    x = x_ref[...]
    cb = cb_ref[...]
    mm = lax.dot_general(x, cb, (((1,), (1,)), ((), ())),
                         preferred_element_type=jnp.float32)
    d = (xsq_ref[...] - 2.0 * mm) + esq_ref[...]

    bmin = jnp.min(d, axis=1, keepdims=True)
    iota = lax.broadcasted_iota(jnp.int32, d.shape, 1)
    masked = jnp.where(d == bmin, iota, jnp.int32(block_k))
    bidx = jnp.min(masked, axis=1, keepdims=True) + k * block_k

    @pl.when(k == 0)
    def _():
        runmin_ref[...] = bmin
        runidx_ref[...] = bidx

    @pl.when(k != 0)
    def _():
        better = bmin < runmin_ref[...]
        runmin_ref[...] = jnp.where(better, bmin, runmin_ref[...])
        runidx_ref[...] = jnp.where(better, bidx, runidx_ref[...])

    @pl.when(k == nk - 1)
    def _():
        idx_ref[...] = runidx_ref[...]
        loss_ref[...] = jnp.sum(runmin_ref[...], keepdims=True) * (1.25 / n_elems)


def _argmin_call(x, cb, xsq, esq, block_k=512):
    m, d = x.shape
    kk = cb.shape[0]
    grid = (kk // block_k,)
    body = functools.partial(_argmin_body, block_k=block_k, n_elems=m * d)
    return pl.pallas_call(
        body,
        grid=grid,
        in_specs=[
            pl.BlockSpec((m, d), lambda k: (0, 0)),
            pl.BlockSpec((block_k, d), lambda k: (k, 0)),
            pl.BlockSpec((m, 1), lambda k: (0, 0)),
            pl.BlockSpec((1, block_k), lambda k: (0, k)),
        ],
        out_specs=[
            pl.BlockSpec((m, 1), lambda k: (0, 0)),
            pl.BlockSpec((1, 1), lambda k: (0, 0)),
        ],
        out_shape=[
            jax.ShapeDtypeStruct((m, 1), jnp.int32),
            jax.ShapeDtypeStruct((1, 1), jnp.float32),
        ],
        scratch_shapes=[
            pltpu.VMEM((m, 1), jnp.float32),
            pltpu.VMEM((m, 1), jnp.int32),
        ],
    )(x, cb, xsq, esq)


def _make_sc_gather(kk, d, b):
    info = plsc.get_sparse_core_info()
    nw = info.num_cores * info.num_subcores
    b_per_w = b // nw
    mesh = plsc.VectorSubcoreMesh(core_axis_name="c", subcore_axis_name="s")

    @functools.partial(
        pl.kernel, mesh=mesh,
        out_type=jax.ShapeDtypeStruct((b, d), jnp.float32),
        scratch_types=[
            pltpu.VMEM((b_per_w,), jnp.int32),
            pltpu.VMEM((b_per_w, d), jnp.float32),
            pltpu.SemaphoreType.DMA,
        ],
    )
    def gather(table_hbm, idx_hbm, out_hbm, idx_v, rows_v, sem):
        wid = lax.axis_index("s") * info.num_cores + lax.axis_index("c")
        base = wid * b_per_w
        pltpu.sync_copy(idx_hbm.at[pl.ds(base, b_per_w)], idx_v)
        pltpu.async_copy(table_hbm.at[idx_v], rows_v, sem).wait()
        pltpu.sync_copy(rows_v, out_hbm.at[pl.ds(base, b_per_w)])

    return gather


def _st_body(z_ref, q_ref, out_ref):
    z = z_ref[...]
    out_ref[...] = z + (q_ref[...] - z)


def _st_call(z2d, quant, block_m=576):
    m, d = z2d.shape
    return pl.pallas_call(
        _st_body,
        grid=(m // block_m,),
        in_specs=[
            pl.BlockSpec((block_m, d), lambda i: (i, 0)),
            pl.BlockSpec((block_m, d), lambda i: (i, 0)),
        ],
        out_specs=pl.BlockSpec((block_m, d), lambda i: (i, 0)),
        out_shape=jax.ShapeDtypeStruct((m, d), jnp.float32),
    )(z2d, quant)


def kernel(z, codebook):
    b, t, d = z.shape
    kk = codebook.shape[0]
    flat = z.reshape(-1, d)
    xsq = jnp.sum(flat ** 2, axis=1, keepdims=True)
    esq = jnp.sum(codebook ** 2, axis=1)[None, :]

    idx2d, loss2d = _argmin_call(flat, codebook, xsq, esq)
    idx = idx2d.reshape(-1)

    quant = _make_sc_gather(kk, d, flat.shape[0])(codebook, idx)

    quant_st = _st_call(flat, quant).reshape(b, t, d)
    return (quant_st, loss2d[0, 0])


# speedup vs baseline: 1.2399x; 1.1775x over previous
import functools

import jax
import jax.numpy as jnp
from jax import lax
from jax.experimental import pallas as pl
from jax.experimental.pallas import tpu as pltpu
from jax.experimental.pallas import tpu_sc as plsc


def _argmin_body(x_ref, cb_ref, xsq_ref, esq_ref, idx_ref, loss_ref,
                 runmin_ref, runidx_ref, *, block_k, n_elems):
    k = pl.program_id(0)
    nk = pl.num_programs(0)

    x = x_ref[...]
    cb = cb_ref[...]
    mm = lax.dot_general(cb, x, (((1,), (1,)), ((), ())),
                         preferred_element_type=jnp.float32)
    d = (xsq_ref[...] - 2.0 * mm) + esq_ref[...]

    bmin = jnp.min(d, axis=0, keepdims=True)
    iota = lax.broadcasted_iota(jnp.int32, d.shape, 0)
    masked = jnp.where(d == bmin, iota, jnp.int32(block_k))
    bidx = jnp.min(masked, axis=0, keepdims=True) + k * block_k

    @pl.when(k == 0)
    def _():
        runmin_ref[...] = bmin
        runidx_ref[...] = bidx

    @pl.when(k != 0)
    def _():
        better = bmin < runmin_ref[...]
        runmin_ref[...] = jnp.where(better, bmin, runmin_ref[...])
        runidx_ref[...] = jnp.where(better, bidx, runidx_ref[...])

    @pl.when(k == nk - 1)
    def _():
        idx_ref[...] = runidx_ref[...]
        loss_ref[...] = jnp.sum(runmin_ref[...], keepdims=True) * (1.25 / n_elems)


def _argmin_call(x, cb, xsq, esq, block_k=512):
    m, d = x.shape
    kk = cb.shape[0]
    grid = (kk // block_k,)
    body = functools.partial(_argmin_body, block_k=block_k, n_elems=m * d)
    return pl.pallas_call(
        body,
        grid=grid,
        in_specs=[
            pl.BlockSpec((m, d), lambda k: (0, 0)),
            pl.BlockSpec((block_k, d), lambda k: (k, 0)),
            pl.BlockSpec((1, m), lambda k: (0, 0)),
            pl.BlockSpec((block_k, 1), lambda k: (k, 0)),
        ],
        out_specs=[
            pl.BlockSpec((1, m), lambda k: (0, 0)),
            pl.BlockSpec((1, 1), lambda k: (0, 0)),
        ],
        out_shape=[
            jax.ShapeDtypeStruct((1, m), jnp.int32),
            jax.ShapeDtypeStruct((1, 1), jnp.float32),
        ],
        scratch_shapes=[
            pltpu.VMEM((1, m), jnp.float32),
            pltpu.VMEM((1, m), jnp.int32),
        ],
    )(x, cb, xsq, esq)


def _make_sc_gather(kk, d, b):
    info = plsc.get_sparse_core_info()
    nw = info.num_cores * info.num_subcores
    b_per_w = b // nw
    mesh = plsc.VectorSubcoreMesh(core_axis_name="c", subcore_axis_name="s")

    @functools.partial(
        pl.kernel, mesh=mesh,
        out_type=jax.ShapeDtypeStruct((b, d), jnp.float32),
        scratch_types=[
            pltpu.VMEM((b_per_w,), jnp.int32),
            pltpu.VMEM((b_per_w, d), jnp.float32),
            pltpu.SemaphoreType.DMA,
        ],
    )
    def gather(table_hbm, idx_hbm, out_hbm, idx_v, rows_v, sem):
        wid = lax.axis_index("s") * info.num_cores + lax.axis_index("c")
        base = wid * b_per_w
        pltpu.sync_copy(idx_hbm.at[pl.ds(base, b_per_w)], idx_v)
        pltpu.async_copy(table_hbm.at[idx_v], rows_v, sem).wait()
        pltpu.sync_copy(rows_v, out_hbm.at[pl.ds(base, b_per_w)])

    return gather


def _st_body(z_ref, q_ref, out_ref):
    z = z_ref[...]
    out_ref[...] = z + (q_ref[...] - z)


def _st_call(z2d, quant, block_m=576):
    m, d = z2d.shape
    return pl.pallas_call(
        _st_body,
        grid=(m // block_m,),
        in_specs=[
            pl.BlockSpec((block_m, d), lambda i: (i, 0)),
            pl.BlockSpec((block_m, d), lambda i: (i, 0)),
        ],
        out_specs=pl.BlockSpec((block_m, d), lambda i: (i, 0)),
        out_shape=jax.ShapeDtypeStruct((m, d), jnp.float32),
    )(z2d, quant)


def kernel(z, codebook):
    b, t, d = z.shape
    kk = codebook.shape[0]
    flat = z.reshape(-1, d)
    xsq = jnp.sum(flat ** 2, axis=1, keepdims=True).reshape(1, -1)
    esq = jnp.sum(codebook ** 2, axis=1)[:, None]

    idx2d, loss2d = _argmin_call(flat, codebook, xsq, esq)
    idx = idx2d.reshape(-1)

    quant = _make_sc_gather(kk, d, flat.shape[0])(codebook, idx)

    quant_st = _st_call(flat, quant).reshape(b, t, d)
    return (quant_st, loss2d[0, 0])


# speedup vs baseline: 1.3406x; 1.0813x over previous
import functools

import jax
import jax.numpy as jnp
from jax import lax
from jax.experimental import pallas as pl
from jax.experimental.pallas import tpu as pltpu
from jax.experimental.pallas import tpu_sc as plsc


def _argmin_body(x_ref, cb_ref, xsq_ref, esq_ref, idx_ref, loss_ref,
                 runmin_ref, runidx_ref, *, block_k, n_elems):
    k = pl.program_id(0)
    nk = pl.num_programs(0)

    x = x_ref[...]
    cb = cb_ref[...]
    mm = lax.dot_general(cb, x, (((1,), (1,)), ((), ())),
                         preferred_element_type=jnp.float32)
    d = (xsq_ref[...] - 2.0 * mm) + esq_ref[...]

    bmin = jnp.min(d, axis=0, keepdims=True)
    iota = lax.broadcasted_iota(jnp.int32, d.shape, 0)
    masked = jnp.where(d == bmin, iota, jnp.int32(block_k))
    bidx = jnp.min(masked, axis=0, keepdims=True) + k * block_k

    @pl.when(k == 0)
    def _():
        runmin_ref[...] = bmin
        runidx_ref[...] = bidx

    @pl.when(k != 0)
    def _():
        better = bmin < runmin_ref[...]
        runmin_ref[...] = jnp.where(better, bmin, runmin_ref[...])
        runidx_ref[...] = jnp.where(better, bidx, runidx_ref[...])

    @pl.when(k == nk - 1)
    def _():
        idx_ref[...] = runidx_ref[...]
        loss_ref[...] = jnp.sum(runmin_ref[...], keepdims=True) * (1.25 / n_elems)


def _argmin_call(x, cb, xsq, esq, block_k=1024):
    m, d = x.shape
    kk = cb.shape[0]
    grid = (kk // block_k,)
    body = functools.partial(_argmin_body, block_k=block_k, n_elems=m * d)
    return pl.pallas_call(
        body,
        grid=grid,
        in_specs=[
            pl.BlockSpec((m, d), lambda k: (0, 0)),
            pl.BlockSpec((block_k, d), lambda k: (k, 0)),
            pl.BlockSpec((1, m), lambda k: (0, 0)),
            pl.BlockSpec((block_k, 1), lambda k: (k, 0)),
        ],
        out_specs=[
            pl.BlockSpec((1, m), lambda k: (0, 0)),
            pl.BlockSpec((1, 1), lambda k: (0, 0)),
        ],
        out_shape=[
            jax.ShapeDtypeStruct((1, m), jnp.int32),
            jax.ShapeDtypeStruct((1, 1), jnp.float32),
        ],
        scratch_shapes=[
            pltpu.VMEM((1, m), jnp.float32),
            pltpu.VMEM((1, m), jnp.int32),
        ],
    )(x, cb, xsq, esq)


def _make_sc_gather(kk, d, b):
    info = plsc.get_sparse_core_info()
    nw = info.num_cores * info.num_subcores
    b_per_w = b // nw
    mesh = plsc.VectorSubcoreMesh(core_axis_name="c", subcore_axis_name="s")

    @functools.partial(
        pl.kernel, mesh=mesh,
        out_type=jax.ShapeDtypeStruct((b, d), jnp.float32),
        scratch_types=[
            pltpu.VMEM((b_per_w,), jnp.int32),
            pltpu.VMEM((b_per_w, d), jnp.float32),
            pltpu.SemaphoreType.DMA,
        ],
    )
    def gather(table_hbm, idx_hbm, out_hbm, idx_v, rows_v, sem):
        wid = lax.axis_index("s") * info.num_cores + lax.axis_index("c")
        base = wid * b_per_w
        pltpu.sync_copy(idx_hbm.at[pl.ds(base, b_per_w)], idx_v)
        pltpu.async_copy(table_hbm.at[idx_v], rows_v, sem).wait()
        pltpu.sync_copy(rows_v, out_hbm.at[pl.ds(base, b_per_w)])

    return gather


def _st_body(z_ref, q_ref, out_ref):
    z = z_ref[...]
    out_ref[...] = z + (q_ref[...] - z)


def _st_call(z2d, quant, block_m=576):
    m, d = z2d.shape
    return pl.pallas_call(
        _st_body,
        grid=(m // block_m,),
        in_specs=[
            pl.BlockSpec((block_m, d), lambda i: (i, 0)),
            pl.BlockSpec((block_m, d), lambda i: (i, 0)),
        ],
        out_specs=pl.BlockSpec((block_m, d), lambda i: (i, 0)),
        out_shape=jax.ShapeDtypeStruct((m, d), jnp.float32),
    )(z2d, quant)


def kernel(z, codebook):
    b, t, d = z.shape
    kk = codebook.shape[0]
    flat = z.reshape(-1, d)
    xsq = jnp.sum(flat ** 2, axis=1, keepdims=True).reshape(1, -1)
    esq = jnp.sum(codebook ** 2, axis=1)[:, None]

    idx2d, loss2d = _argmin_call(flat, codebook, xsq, esq)
    idx = idx2d.reshape(-1)

    quant = _make_sc_gather(kk, d, flat.shape[0])(codebook, idx)

    quant_st = _st_call(flat, quant).reshape(b, t, d)
    return (quant_st, loss2d[0, 0])


# speedup vs baseline: 1.4581x; 1.0876x over previous
import functools

import jax
import jax.numpy as jnp
from jax import lax
from jax.experimental import pallas as pl
from jax.experimental.pallas import tpu as pltpu
from jax.experimental.pallas import tpu_sc as plsc


def _argmin_body(x_ref, cb_ref, xsq_ref, esq_ref, idx_ref, loss_ref,
                 runmin_ref, runidx_ref, *, block_k, n_elems):
    k = pl.program_id(0)
    nk = pl.num_programs(0)

    x = x_ref[...]
    cb = cb_ref[...]
    mm = lax.dot_general(cb, x, (((1,), (1,)), ((), ())),
                         preferred_element_type=jnp.float32)
    d = (xsq_ref[...] - 2.0 * mm) + esq_ref[...]

    bmin = jnp.min(d, axis=0, keepdims=True)
    iota = lax.broadcasted_iota(jnp.int32, d.shape, 0)
    masked = jnp.where(d == bmin, iota, jnp.int32(block_k))
    bidx = jnp.min(masked, axis=0, keepdims=True) + k * block_k

    @pl.when(k == 0)
    def _():
        runmin_ref[...] = bmin
        runidx_ref[...] = bidx

    @pl.when(k != 0)
    def _():
        better = bmin < runmin_ref[...]
        runmin_ref[...] = jnp.where(better, bmin, runmin_ref[...])
        runidx_ref[...] = jnp.where(better, bidx, runidx_ref[...])

    @pl.when(k == nk - 1)
    def _():
        idx_ref[...] = runidx_ref[...]
        loss_ref[...] = jnp.sum(runmin_ref[...], keepdims=True) * (1.25 / n_elems)


def _argmin_call(x, cb, xsq, esq, block_k=1024):
    m, d = x.shape
    kk = cb.shape[0]
    grid = (kk // block_k,)
    body = functools.partial(_argmin_body, block_k=block_k, n_elems=m * d)
    return pl.pallas_call(
        body,
        grid=grid,
        in_specs=[
            pl.BlockSpec((m, d), lambda k: (0, 0)),
            pl.BlockSpec((block_k, d), lambda k: (k, 0)),
            pl.BlockSpec((1, m), lambda k: (0, 0)),
            pl.BlockSpec((block_k, 1), lambda k: (k, 0)),
        ],
        out_specs=[
            pl.BlockSpec((1, m), lambda k: (0, 0)),
            pl.BlockSpec((1, 1), lambda k: (0, 0)),
        ],
        out_shape=[
            jax.ShapeDtypeStruct((1, m), jnp.int32),
            jax.ShapeDtypeStruct((1, 1), jnp.float32),
        ],
        scratch_shapes=[
            pltpu.VMEM((1, m), jnp.float32),
            pltpu.VMEM((1, m), jnp.int32),
        ],
    )(x, cb, xsq, esq)


def _make_sc_gather(kk, d, b):
    info = plsc.get_sparse_core_info()
    nw = info.num_cores * info.num_subcores
    b_per_w = b // nw
    mesh = plsc.VectorSubcoreMesh(core_axis_name="c", subcore_axis_name="s")

    @functools.partial(
        pl.kernel, mesh=mesh,
        out_type=jax.ShapeDtypeStruct((b, d), jnp.float32),
        scratch_types=[
            pltpu.VMEM((b_per_w,), jnp.int32),
            pltpu.VMEM((b_per_w, d), jnp.float32),
            pltpu.SemaphoreType.DMA,
        ],
    )
    def gather(table_hbm, idx_hbm, out_hbm, idx_v, rows_v, sem):
        wid = lax.axis_index("s") * info.num_cores + lax.axis_index("c")
        base = wid * b_per_w
        pltpu.sync_copy(idx_hbm.at[pl.ds(base, b_per_w)], idx_v)
        pltpu.async_copy(table_hbm.at[idx_v], rows_v, sem).wait()
        pltpu.sync_copy(rows_v, out_hbm.at[pl.ds(base, b_per_w)])

    return gather


def kernel(z, codebook):
    b, t, d = z.shape
    kk = codebook.shape[0]
    flat = z.reshape(-1, d)
    xsq = jnp.sum(flat ** 2, axis=1, keepdims=True).reshape(1, -1)
    esq = jnp.sum(codebook ** 2, axis=1)[:, None]

    idx2d, loss2d = _argmin_call(flat, codebook, xsq, esq)
    idx = idx2d.reshape(-1)

    quant_st = _make_sc_gather(kk, d, flat.shape[0])(codebook, idx)
    return (quant_st.reshape(b, t, d), loss2d[0, 0])


# speedup vs baseline: 1.4839x; 1.0177x over previous
import functools

import jax
import jax.numpy as jnp
from jax import lax
from jax.experimental import pallas as pl
from jax.experimental.pallas import tpu as pltpu
from jax.experimental.pallas import tpu_sc as plsc


def _argmin_body(x2_ref, cb_ref, xsq_ref, esq_ref, iof_ref, idx_ref, loss_ref,
                 runmin_ref, runidx_ref, *, n_elems):
    k = pl.program_id(0)
    nk = pl.num_programs(0)

    x2 = x2_ref[...]
    cb = cb_ref[...]
    mm2 = lax.dot_general(cb, x2, (((1,), (1,)), ((), ())),
                          preferred_element_type=jnp.float32)
    d = (xsq_ref[...] - mm2) + esq_ref[...]

    bmin = jnp.min(d, axis=0, keepdims=True)
    masked = jnp.where(d == bmin, iof_ref[...], jnp.float32(65536.0))
    bidx = jnp.min(masked, axis=0, keepdims=True)

    @pl.when(k == 0)
    def _():
        runmin_ref[...] = bmin
        runidx_ref[...] = bidx

    @pl.when(k != 0)
    def _():
        better = bmin < runmin_ref[...]
        runmin_ref[...] = jnp.where(better, bmin, runmin_ref[...])
        runidx_ref[...] = jnp.where(better, bidx, runidx_ref[...])

    @pl.when(k == nk - 1)
    def _():
        idx_ref[...] = runidx_ref[...].astype(jnp.int32)
        loss_ref[...] = jnp.sum(runmin_ref[...], keepdims=True) * (1.25 / n_elems)


def _argmin_call(x2, cb, xsq, esq, block_k=1024):
    m, d = x2.shape
    kk = cb.shape[0]
    grid = (kk // block_k,)
    iof = jnp.arange(kk, dtype=jnp.float32)[:, None]
    body = functools.partial(_argmin_body, n_elems=m * d)
    return pl.pallas_call(
        body,
        grid=grid,
        in_specs=[
            pl.BlockSpec((m, d), lambda k: (0, 0)),
            pl.BlockSpec((block_k, d), lambda k: (k, 0)),
            pl.BlockSpec((1, m), lambda k: (0, 0)),
            pl.BlockSpec((block_k, 1), lambda k: (k, 0)),
            pl.BlockSpec((block_k, 1), lambda k: (k, 0)),
        ],
        out_specs=[
            pl.BlockSpec((1, m), lambda k: (0, 0)),
            pl.BlockSpec((1, 1), lambda k: (0, 0)),
        ],
        out_shape=[
            jax.ShapeDtypeStruct((1, m), jnp.int32),
            jax.ShapeDtypeStruct((1, 1), jnp.float32),
        ],
        scratch_shapes=[
            pltpu.VMEM((1, m), jnp.float32),
            pltpu.VMEM((1, m), jnp.float32),
        ],
    )(x2, cb, xsq, esq, iof)


def _make_sc_gather(kk, d, b):
    info = plsc.get_sparse_core_info()
    nw = info.num_cores * info.num_subcores
    b_per_w = b // nw
    mesh = plsc.VectorSubcoreMesh(core_axis_name="c", subcore_axis_name="s")

    @functools.partial(
        pl.kernel, mesh=mesh,
        out_type=jax.ShapeDtypeStruct((b, d), jnp.float32),
        scratch_types=[
            pltpu.VMEM((b_per_w,), jnp.int32),
            pltpu.VMEM((b_per_w, d), jnp.float32),
            pltpu.SemaphoreType.DMA,
        ],
    )
    def gather(table_hbm, idx_hbm, out_hbm, idx_v, rows_v, sem):
        wid = lax.axis_index("s") * info.num_cores + lax.axis_index("c")
        base = wid * b_per_w
        pltpu.sync_copy(idx_hbm.at[pl.ds(base, b_per_w)], idx_v)
        pltpu.async_copy(table_hbm.at[idx_v], rows_v, sem).wait()
        pltpu.sync_copy(rows_v, out_hbm.at[pl.ds(base, b_per_w)])

    return gather


def kernel(z, codebook):
    b, t, d = z.shape
    kk = codebook.shape[0]
    flat = z.reshape(-1, d)
    xsq = jnp.sum(flat ** 2, axis=1, keepdims=True).reshape(1, -1)
    esq = jnp.sum(codebook ** 2, axis=1)[:, None]

    idx2d, loss2d = _argmin_call(flat * 2.0, codebook, xsq, esq)
    idx = idx2d.reshape(-1)

    quant_st = _make_sc_gather(kk, d, flat.shape[0])(codebook, idx)
    return (quant_st.reshape(b, t, d), loss2d[0, 0])


# speedup vs baseline: 1.6705x; 1.1258x over previous
import functools

import jax
import jax.numpy as jnp
from jax import lax
from jax.experimental import pallas as pl
from jax.experimental.pallas import tpu as pltpu
from jax.experimental.pallas import tpu_sc as plsc


def _argmin_body(x2_ref, cb_ref, xsq_ref, esq_ref, idx_ref, loss_ref,
                 runmin_ref, runidx_ref, *, block_k, n_elems):
    k = pl.program_id(0)
    nk = pl.num_programs(0)

    x2 = x2_ref[...]
    cb = cb_ref[...]
    mm2 = lax.dot_general(cb, x2, (((1,), (1,)), ((), ())),
                          preferred_element_type=jnp.float32)
    d = (xsq_ref[...] - mm2) + esq_ref[...]

    bmin = jnp.min(d, axis=0, keepdims=True)
    bidx = (jnp.argmin(d, axis=0).astype(jnp.float32)[None, :]
            + (k * block_k).astype(jnp.float32))

    @pl.when(k == 0)
    def _():
        runmin_ref[...] = bmin
        runidx_ref[...] = bidx

    @pl.when(k != 0)
    def _():
        better = bmin < runmin_ref[...]
        runmin_ref[...] = jnp.where(better, bmin, runmin_ref[...])
        runidx_ref[...] = jnp.where(better, bidx, runidx_ref[...])

    @pl.when(k == nk - 1)
    def _():
        idx_ref[...] = runidx_ref[...].astype(jnp.int32)
        loss_ref[...] = jnp.sum(runmin_ref[...], keepdims=True) * (1.25 / n_elems)


def _argmin_call(x2, cb, xsq, esq, block_k=1024):
    m, d = x2.shape
    kk = cb.shape[0]
    grid = (kk // block_k,)
    body = functools.partial(_argmin_body, block_k=block_k, n_elems=m * d)
    return pl.pallas_call(
        body,
        grid=grid,
        in_specs=[
            pl.BlockSpec((m, d), lambda k: (0, 0)),
            pl.BlockSpec((block_k, d), lambda k: (k, 0)),
            pl.BlockSpec((1, m), lambda k: (0, 0)),
            pl.BlockSpec((block_k, 1), lambda k: (k, 0)),
        ],
        out_specs=[
            pl.BlockSpec((1, m), lambda k: (0, 0)),
            pl.BlockSpec((1, 1), lambda k: (0, 0)),
        ],
        out_shape=[
            jax.ShapeDtypeStruct((1, m), jnp.int32),
            jax.ShapeDtypeStruct((1, 1), jnp.float32),
        ],
        scratch_shapes=[
            pltpu.VMEM((1, m), jnp.float32),
            pltpu.VMEM((1, m), jnp.float32),
        ],
    )(x2, cb, xsq, esq)


def _make_sc_gather(kk, d, b):
    info = plsc.get_sparse_core_info()
    nw = info.num_cores * info.num_subcores
    b_per_w = b // nw
    mesh = plsc.VectorSubcoreMesh(core_axis_name="c", subcore_axis_name="s")

    @functools.partial(
        pl.kernel, mesh=mesh,
        out_type=jax.ShapeDtypeStruct((b, d), jnp.float32),
        scratch_types=[
            pltpu.VMEM((b_per_w,), jnp.int32),
            pltpu.VMEM((b_per_w, d), jnp.float32),
            pltpu.SemaphoreType.DMA,
        ],
    )
    def gather(table_hbm, idx_hbm, out_hbm, idx_v, rows_v, sem):
        wid = lax.axis_index("s") * info.num_cores + lax.axis_index("c")
        base = wid * b_per_w
        pltpu.sync_copy(idx_hbm.at[pl.ds(base, b_per_w)], idx_v)
        pltpu.async_copy(table_hbm.at[idx_v], rows_v, sem).wait()
        pltpu.sync_copy(rows_v, out_hbm.at[pl.ds(base, b_per_w)])

    return gather


def kernel(z, codebook):
    b, t, d = z.shape
    kk = codebook.shape[0]
    flat = z.reshape(-1, d)
    xsq = jnp.sum(flat ** 2, axis=1, keepdims=True).reshape(1, -1)
    esq = jnp.sum(codebook ** 2, axis=1)[:, None]

    idx2d, loss2d = _argmin_call(flat * 2.0, codebook, xsq, esq)
    idx = idx2d.reshape(-1)

    quant_st = _make_sc_gather(kk, d, flat.shape[0])(codebook, idx)
    return (quant_st.reshape(b, t, d), loss2d[0, 0])
